# baseline (speedup 1.0000x reference)
import functools

import jax
import jax.numpy as jnp
from jax import lax
from jax.experimental import pallas as pl
from jax.experimental.pallas import tpu as pltpu
from jax.experimental.pallas import tpu_sc as plsc

D_MODEL = 64

NC = 2
NS = 16
NW = NC * NS

G = 128
K = 4


@functools.partial(jax.jit, static_argnames=("b_total",))
def _gather_rows(weight, idx_grp, b_total):
  n_groups = idx_grp.shape[1]
  b_per_w = n_groups * G
  n_outer = n_groups // K

  mesh = plsc.VectorSubcoreMesh(
      core_axis_name="c", subcore_axis_name="s", num_cores=NC, num_subcores=NS
  )

  @functools.partial(
      pl.kernel,
      mesh=mesh,
      compiler_params=pltpu.CompilerParams(use_tc_tiling_on_sc=False),
      out_type=jax.ShapeDtypeStruct((b_total, D_MODEL), jnp.float32),
      scratch_types=(
          [pltpu.VMEM((n_groups, G), jnp.int32),
           pltpu.VMEM((K, G, D_MODEL), jnp.float32)]
          + [pltpu.SemaphoreType.DMA] * (2 * K)
      ),
  )
  def k(table_hbm, idx_hbm, out_hbm, idx_v, rows_v, *sems):
    gsem = sems[:K]
    osem = sems[K:]
    wid = lax.axis_index("s") * NC + lax.axis_index("c")
    base = wid * b_per_w
    pltpu.sync_copy(idx_hbm.at[wid], idx_v)

    def outer(i, carry):
      g0 = i * K
      for b in range(K):
        @pl.when(i > 0)
        def _wait_out():
          pltpu.make_async_copy(
              rows_v.at[b],
              out_hbm.at[pl.ds(base + (g0 + b - K) * G, G)],
              osem[b],
          ).wait()

        pltpu.async_copy(table_hbm.at[idx_v.at[g0 + b]], rows_v.at[b],
                         gsem[b])
      for b in range(K):
        pltpu.make_async_copy(table_hbm.at[idx_v.at[g0 + b]], rows_v.at[b],
                              gsem[b]).wait()
        pltpu.async_copy(rows_v.at[b],
                         out_hbm.at[pl.ds(base + (g0 + b) * G, G)],
                         osem[b])
      return carry

    lax.fori_loop(0, n_outer, outer, 0)
    for b in range(K):
      pltpu.make_async_copy(
          rows_v.at[b],
          out_hbm.at[pl.ds(base + (n_outer * K - K + b) * G, G)],
          osem[b],
      ).wait()

  return k(weight, idx_grp)


def kernel(x, weight):
  shape = x.shape + (D_MODEL,)
  idx = x.reshape(-1).astype(jnp.int32)
  b_total = idx.shape[0]
  n_groups = b_total // (NW * G)
  idx_grp = idx.reshape(NW, n_groups, G)
  out = _gather_rows(weight, idx_grp, b_total)
  return out.reshape(shape)


# speedup vs baseline: 1.0004x; 1.0004x over previous
import functools

import jax
import jax.numpy as jnp
from jax import lax
from jax.experimental import pallas as pl
from jax.experimental.pallas import tpu as pltpu
from jax.experimental.pallas import tpu_sc as plsc

D_MODEL = 64

NC = 2
NS = 16
NW = NC * NS

G = 128
SUPER = 4
NB = 2


@functools.partial(jax.jit, static_argnames=("b_total",))
def _gather_rows(weight, idx_grp, b_total):
  n_groups = idx_grp.shape[1]
  b_per_w = n_groups * G
  rows_sup = SUPER * G
  n_outer = n_groups // (SUPER * NB)

  mesh = plsc.VectorSubcoreMesh(
      core_axis_name="c", subcore_axis_name="s", num_cores=NC, num_subcores=NS
  )

  @functools.partial(
      pl.kernel,
      mesh=mesh,
      compiler_params=pltpu.CompilerParams(use_tc_tiling_on_sc=False),
      out_type=jax.ShapeDtypeStruct((b_total, D_MODEL), jnp.float32),
      scratch_types=(
          [pltpu.VMEM((n_groups, G), jnp.int32),
           pltpu.VMEM((NB, rows_sup, D_MODEL), jnp.float32)]
          + [pltpu.SemaphoreType.DMA] * (NB * SUPER + NB)
      ),
  )
  def k(table_hbm, idx_hbm, out_hbm, idx_v, rows_v, *sems):
    gsem = sems[:NB * SUPER]
    osem = sems[NB * SUPER:]
    wid = lax.axis_index("s") * NC + lax.axis_index("c")
    base = wid * b_per_w
    pltpu.sync_copy(idx_hbm.at[wid], idx_v)

    def outer(i, carry):
      g0 = i * NB * SUPER
      for o in range(NB):
        @pl.when(i > 0)
        def _wait_out():
          pltpu.make_async_copy(
              rows_v.at[o],
              out_hbm.at[pl.ds(base + (g0 + o * SUPER - NB * SUPER) * G,
                               rows_sup)],
              osem[o],
          ).wait()

        for j in range(SUPER):
          pltpu.async_copy(table_hbm.at[idx_v.at[g0 + o * SUPER + j]],
                           rows_v.at[o, pl.ds(j * G, G)],
                           gsem[o * SUPER + j])
      for o in range(NB):
        for j in range(SUPER):
          pltpu.make_async_copy(table_hbm.at[idx_v.at[g0 + o * SUPER + j]],
                                rows_v.at[o, pl.ds(j * G, G)],
                                gsem[o * SUPER + j]).wait()
        pltpu.async_copy(rows_v.at[o],
                         out_hbm.at[pl.ds(base + (g0 + o * SUPER) * G,
                                          rows_sup)],
                         osem[o])
      return carry

    lax.fori_loop(0, n_outer, outer, 0)
    for o in range(NB):
      pltpu.make_async_copy(
          rows_v.at[o],
          out_hbm.at[pl.ds(base + ((n_outer - 1) * NB * SUPER + o * SUPER) * G,
                           rows_sup)],
          osem[o],
      ).wait()

  return k(weight, idx_grp)


def kernel(x, weight):
  shape = x.shape + (D_MODEL,)
  idx = x.reshape(-1).astype(jnp.int32)
  b_total = idx.shape[0]
  n_groups = b_total // (NW * G)
  idx_grp = idx.reshape(NW, n_groups, G)
  out = _gather_rows(weight, idx_grp, b_total)
  return out.reshape(shape)
